# baseline (speedup 1.0000x reference)
import jax
import jax.numpy as jnp
from jax.experimental import pallas as pl

_BLOCK = 1000
_NSEL = 64


def _body(x_ref, o_ref):
    i = pl.program_id(0)
    xb = x_ref[...]
    rid = jax.lax.broadcasted_iota(jnp.int32, xb.shape, 0)
    mask = (rid == 0) & (i < _NSEL)
    o_ref[...] = jnp.where(mask, jnp.maximum(xb, 0.0), xb)


def kernel(x):
    rows, cols = x.shape
    grid = rows // _BLOCK
    return pl.pallas_call(
        _body,
        grid=(grid,),
        in_specs=[pl.BlockSpec((_BLOCK, cols), lambda i: (i, 0))],
        out_specs=pl.BlockSpec((_BLOCK, cols), lambda i: (i, 0)),
        out_shape=jax.ShapeDtypeStruct(x.shape, x.dtype),
    )(x)
